# baseline (speedup 1.0000x reference)
import jax
import jax.numpy as jnp
from jax import lax
from jax.experimental import pallas as pl
from jax.experimental.pallas import tpu as pltpu
from jax.experimental.pallas import tpu_sc as plsc

_N = 100000
_G = 512
_NC = 2
_NS = 16
_L = 16

_PER_W = 6256
_LAST_W = _N - (_NS - 1) * _PER_W
_VECS = _PER_W // _L

_FPW = 18752
_FLAST = _N * 3 - (_NS - 1) * _FPW
_FB = 2048


def _sc_body(pos_hbm, batch_hbm, energy_hbm, forces_hbm,
             pos_v, b_v, z_v, r_v, f_v, sem, acc_sh):
  c = lax.axis_index("c")
  s = lax.axis_index("s")

  @pl.when(c == 0)
  def _():
    base = s * _PER_W
    count = jnp.where(s == _NS - 1, _LAST_W, _PER_W)

    @pl.when(s < _NS - 1)
    def _():
      pltpu.async_copy(
          pos_hbm.at[pl.ds(base * 3, _PER_W * 3)], pos_v, sem)
      pltpu.async_copy(
          batch_hbm.at[pl.ds(base, _PER_W)], b_v, sem)

    @pl.when(s == _NS - 1)
    def _():
      pltpu.async_copy(
          pos_hbm.at[pl.ds(base * 3, _LAST_W * 3)],
          pos_v.at[pl.ds(0, _LAST_W * 3)], sem)
      pltpu.async_copy(
          batch_hbm.at[pl.ds(base, _LAST_W)],
          b_v.at[pl.ds(0, _LAST_W)], sem)

    def _zfill(i, _):
      z_v[pl.ds(i * _L, _L)] = jnp.zeros((_L,), jnp.float32)
      return 0
    lax.fori_loop(0, _G // _L, _zfill, 0)

    @pl.when(s < _NS - 1)
    def _():
      pltpu.make_async_copy(
          pos_hbm.at[pl.ds(base * 3, _PER_W * 3)], pos_v, sem).wait()
      pltpu.make_async_copy(
          batch_hbm.at[pl.ds(base, _PER_W)], b_v, sem).wait()

    @pl.when(s == _NS - 1)
    def _():
      pltpu.make_async_copy(
          pos_hbm.at[pl.ds(base * 3, _LAST_W * 3)],
          pos_v.at[pl.ds(0, _LAST_W * 3)], sem).wait()
      pltpu.make_async_copy(
          batch_hbm.at[pl.ds(base, _LAST_W)],
          b_v.at[pl.ds(0, _LAST_W)], sem).wait()

    lanes = lax.iota(jnp.int32, _L)
    lane0 = lanes == 0

    def _step(i, _):
      off = i * _L
      rows = off + lanes
      valid = rows < count
      rows_c = jnp.where(valid, rows, 0)
      y = plsc.load_gather(pos_v, [rows_c * 3 + 1])
      e = jnp.abs(y) * jnp.float32(0.1)
      e = jnp.where(valid, e, jnp.float32(0.0))
      b = b_v[pl.ds(off, _L)]
      b = jnp.where(valid, b, _G - 1)
      mn = jnp.min(b)
      uniform = mn == jnp.max(b)

      @pl.when(uniform)
      def _():
        t = jnp.sum(e)
        plsc.addupdate_scatter(
            z_v, [jnp.zeros((_L,), jnp.int32) + mn],
            jnp.zeros((_L,), jnp.float32) + t, mask=lane0)

      @pl.when(jnp.logical_not(uniform))
      def _():
        for j in range(_L):
          plsc.addupdate_scatter(z_v, [b], e, mask=lanes == j)
      return 0
    lax.fori_loop(0, _VECS, _step, 0)

    pltpu.sync_copy(z_v, acc_sh.at[pl.ds(s * _G, _G)])

    plsc.subcore_barrier()

    _W = _G // _NS
    for k in range(_NS):
      pltpu.async_copy(acc_sh.at[pl.ds(k * _G + s * _W, _W)],
                       r_v.at[pl.ds(k * _W, _W)], sem)
    for k in range(_NS):
      pltpu.make_async_copy(acc_sh.at[pl.ds(k * _G + s * _W, _W)],
                            r_v.at[pl.ds(k * _W, _W)], sem).wait()
    lo = jnp.zeros((_L,), jnp.float32)
    hi = jnp.zeros((_L,), jnp.float32)
    for k in range(_NS):
      lo = lo + r_v[pl.ds(k * _W, _L)]
      hi = hi + r_v[pl.ds(k * _W + _L, _L)]
    z_v[pl.ds(0, _L)] = lo
    z_v[pl.ds(_L, _L)] = hi
    pltpu.sync_copy(z_v.at[pl.ds(0, _W)], energy_hbm.at[pl.ds(s * _W, _W)])

  @pl.when(c == 1)
  def _():
    def _ffill(i, _):
      f_v[pl.ds(i * _L, _L)] = jnp.full((_L,), 0.05, jnp.float32)
      return 0
    lax.fori_loop(0, _FB // _L, _ffill, 0)

    fbase = s * _FPW
    copies = []
    for j in range(9):
      copies.append(pltpu.make_async_copy(
          f_v, forces_hbm.at[pl.ds(fbase + j * _FB, _FB)], sem))

    for cp in copies:
      cp.start()

    @pl.when(s < _NS - 1)
    def _():
      tail = _FPW - 9 * _FB
      pltpu.sync_copy(f_v.at[pl.ds(0, tail)],
                      forces_hbm.at[pl.ds(fbase + 9 * _FB, tail)])

    @pl.when(s == _NS - 1)
    def _():
      tail = _FLAST - 9 * _FB
      pltpu.sync_copy(f_v.at[pl.ds(0, tail)],
                      forces_hbm.at[pl.ds(fbase + 9 * _FB, tail)])

    for cp in copies:
      cp.wait()


def kernel(positions, batch):
  pos_flat = positions.reshape(-1)

  mesh = plsc.VectorSubcoreMesh(
      core_axis_name="c", subcore_axis_name="s",
      num_cores=_NC, num_subcores=_NS)
  sc = pl.kernel(
      _sc_body,
      out_type=(
          jax.ShapeDtypeStruct((_G,), jnp.float32),
          jax.ShapeDtypeStruct((_N * 3,), jnp.float32),
      ),
      mesh=mesh,
      compiler_params=pltpu.CompilerParams(needs_layout_passes=False),
      scratch_types=[
          pltpu.VMEM((_PER_W * 3,), jnp.float32),
          pltpu.VMEM((_PER_W,), jnp.int32),
          pltpu.VMEM((_G,), jnp.float32),
          pltpu.VMEM((_G,), jnp.float32),
          pltpu.VMEM((_FB,), jnp.float32),
          pltpu.SemaphoreType.DMA,
          pltpu.VMEM_SHARED((_NS * _G,), jnp.float32),
      ],
  )
  energies, forces_flat = sc(pos_flat, batch)

  return (energies.reshape(_G, 1), forces_flat.reshape(_N, 3))
